# baseline (speedup 1.0000x reference)
import dataclasses
import functools

import jax
import jax.numpy as jnp
from jax import lax
from jax.experimental import pallas as pl
from jax.experimental.pallas import tpu as pltpu
from jax.experimental.pallas import tpu_sc as plsc

F32 = jnp.float32
BF16 = jnp.bfloat16
I32 = jnp.int32

_TILE = 256
_TB = 256
_NW = 32
_LG2TILE = 8


def _sc_compiler_params():
    cp = pltpu.CompilerParams()
    if "needs_layout_passes" in pltpu.CompilerParams.__dataclass_fields__:
        cp = dataclasses.replace(cp, needs_layout_passes=False)
    return cp


def _router_body(x_ref, wr_ref, w0_ref, w1_ref, mrg_ref, te_ref, counts_ref,
                 carry_ref):
    step = pl.program_id(0)

    @pl.when(step == 0)
    def _():
        carry_ref[...] = jnp.zeros_like(carry_ref)

    x = x_ref[...]
    logits = lax.dot_general(x, wr_ref[...], (((1,), (1,)), ((), ())),
                             preferred_element_type=F32)
    m = jnp.max(logits, axis=1, keepdims=True)
    p = jnp.exp(logits - m)
    probs = p / jnp.sum(p, axis=1, keepdims=True)
    ne = probs.shape[1]
    cols = lax.broadcasted_iota(I32, probs.shape, 1)
    w0 = jnp.max(probs, axis=1, keepdims=True)
    i0 = jnp.min(jnp.where(probs == w0, cols, ne), axis=1, keepdims=True)
    probs2 = jnp.where(cols == i0, jnp.float32(-1.0), probs)
    w1 = jnp.max(probs2, axis=1, keepdims=True)
    i1 = jnp.min(jnp.where(probs2 == w1, cols, ne), axis=1, keepdims=True)
    s = w0 + w1 + 1e-9
    w0_ref[...] = jnp.broadcast_to(w0 / s, w0_ref.shape)
    w1_ref[...] = jnp.broadcast_to(w1 / s, w1_ref.shape)

    tb = x.shape[0]
    oh0 = (lax.broadcasted_iota(I32, (tb, ne), 1) == i0).astype(F32)
    oh1 = (lax.broadcasted_iota(I32, (tb, ne), 1) == i1).astype(F32)
    ohs = jnp.concatenate([oh0, oh1], axis=0)
    ohf = ohs.astype(BF16)
    rows = lax.broadcasted_iota(I32, (2 * tb, 2 * tb), 0)
    colsq = lax.broadcasted_iota(I32, (2 * tb, 2 * tb), 1)
    tri = (rows >= colsq).astype(BF16)
    prefix = lax.dot_general(tri, ohf, (((1,), (0,)), ((), ())),
                             preferred_element_type=F32)
    carry = carry_ref[...].astype(F32)
    rank = jnp.sum((prefix + carry) * ohs, axis=1, keepdims=True) - 1.0
    rank = rank.astype(I32)
    r0 = rank[:tb]
    r1 = rank[tb:]

    lanes = lax.broadcasted_iota(I32, (tb, 128), 1)
    mrg = jnp.where(
        lanes < 8, jnp.broadcast_to(i0, (tb, 128)),
        jnp.where(lanes < 16, jnp.broadcast_to(i1, (tb, 128)),
                  jnp.where(lanes < 24, jnp.broadcast_to(r0, (tb, 128)),
                            jnp.broadcast_to(r1, (tb, 128)))))
    mrg_ref[...] = mrg

    new_counts = carry_ref[...] + jnp.sum(ohs, axis=0,
                                          keepdims=True).astype(I32)
    carry_ref[...] = new_counts
    counts_ref[...] = new_counts

    sizes = lax.shift_left(
        lax.shift_right_logical(new_counts + (_TILE - 1), _LG2TILE),
        _LG2TILE).astype(F32)
    tri16 = (lax.broadcasted_iota(I32, (ne, ne), 0)
             <= lax.broadcasted_iota(I32, (ne, ne), 1)).astype(F32)
    tcum = lax.dot_general(sizes, tri16, (((1,), (0,)), ((), ())),
                           preferred_element_type=F32)
    nt = te_ref.shape[0]
    tile_starts = lax.broadcasted_iota(I32, (nt, ne), 0) * _TILE
    te = jnp.sum((tile_starts >= jnp.broadcast_to(tcum.astype(I32),
                                                  (nt, ne))).astype(I32),
                 axis=1, keepdims=True)
    te = jnp.minimum(te, ne - 1)
    te_ref[...] = jnp.broadcast_to(te, te_ref.shape)


def _shared_body(x_ref, sg_ref, su_ref, sd_ref, o_ref):
    xb = x_ref[...].astype(BF16)
    g = lax.dot_general(xb, sg_ref[...].astype(BF16),
                        (((1,), (1,)), ((), ())), preferred_element_type=F32)
    u = lax.dot_general(xb, su_ref[...].astype(BF16),
                        (((1,), (1,)), ((), ())), preferred_element_type=F32)
    h = (g * jax.nn.sigmoid(g) * u).astype(BF16)
    ms = sd_ref.shape[2]
    y0 = lax.dot_general(h[:, :ms], sd_ref[0].astype(BF16),
                         (((1,), (1,)), ((), ())), preferred_element_type=F32)
    y1 = lax.dot_general(h[:, ms:], sd_ref[1].astype(BF16),
                         (((1,), (1,)), ((), ())), preferred_element_type=F32)
    o_ref[...] = y0 + y1


def _cast_body(wg_ref, wu_ref, wd_ref, og_ref, ou_ref, od_ref):
    og_ref[...] = wg_ref[...].astype(BF16)
    ou_ref[...] = wu_ref[...].astype(BF16)
    od_ref[...] = wd_ref[...].astype(BF16)


def _expert_body(eid_ref, xs_ref, wg_ref, wu_ref, wd_ref, o_ref):
    eid = eid_ref[pl.program_id(0), 0]
    xb = xs_ref[...].astype(BF16)
    g = lax.dot_general(xb, wg_ref[eid],
                        (((1,), (1,)), ((), ())), preferred_element_type=F32)
    u = lax.dot_general(xb, wu_ref[eid],
                        (((1,), (1,)), ((), ())), preferred_element_type=F32)
    h = (g * jax.nn.sigmoid(g) * u).astype(BF16)
    o_ref[...] = lax.dot_general(h, wd_ref[eid],
                                 (((1,), (1,)), ((), ())),
                                 preferred_element_type=F32)


def _combine_body(sh_ref, y0_ref, y1_ref, w0_ref, w1_ref, o_ref):
    o_ref[...] = (sh_ref[...]
                  + w0_ref[:, 0:1] * y0_ref[...]
                  + w1_ref[:, 0:1] * y1_ref[...])


def _dispatch_tokens(flat, mrg, counts, npad):
    t, d = flat.shape
    ne = counts.shape[1]
    ch = t // _NW
    mesh = plsc.VectorSubcoreMesh(core_axis_name="c", subcore_axis_name="s")

    @functools.partial(
        pl.kernel,
        out_type=[
            jax.ShapeDtypeStruct((npad, d), flat.dtype),
            jax.ShapeDtypeStruct((2 * t,), I32),
        ],
        mesh=mesh,
        scratch_types=[
            pltpu.VMEM((ne,), I32),
            pltpu.VMEM((ne,), I32),
            pltpu.VMEM((ch, 128), I32),
            pltpu.VMEM((ch,), I32),
            pltpu.VMEM((ch,), I32),
            pltpu.VMEM((ch, d), flat.dtype),
            pltpu.SemaphoreType.DMA,
            pltpu.SemaphoreType.DMA,
        ],
        compiler_params=_sc_compiler_params(),
    )
    def k(x_hbm, mrg_hbm, cnt_hbm, out_hbm, pos_hbm,
          cnt_v, gb_v, mrg_v, i0_v, i1_v, rows_v, sem, xsem):
        wid = lax.axis_index("c") * 16 + lax.axis_index("s")
        base = wid * ch
        xcp = pltpu.async_copy(x_hbm.at[pl.ds(base, ch)], rows_v, xsem)
        pltpu.sync_copy(cnt_hbm.at[0], cnt_v)
        pltpu.sync_copy(mrg_hbm.at[pl.ds(base, ch)], mrg_v)
        c = cnt_v[...]
        sizes = lax.shift_left(
            lax.shift_right_logical(c + (_TILE - 1), _LG2TILE), _LG2TILE)
        pref = plsc.cumsum(sizes)
        gb_v[...] = pref - sizes
        zero16 = lax.broadcasted_iota(I32, (16,), 0) * 0
        for j in range(ch // 16):
            rowsel = lax.broadcasted_iota(I32, (16,), 0) + (j * 16)
            e0v = plsc.load_gather(mrg_v, [rowsel, zero16])
            e1v = plsc.load_gather(mrg_v, [rowsel, zero16 + 8])
            r0v = plsc.load_gather(mrg_v, [rowsel, zero16 + 16])
            r1v = plsc.load_gather(mrg_v, [rowsel, zero16 + 24])
            i0_v[pl.ds(j * 16, 16)] = plsc.load_gather(gb_v, [e0v]) + r0v
            i1_v[pl.ds(j * 16, 16)] = plsc.load_gather(gb_v, [e1v]) + r1v
        xcp.wait()
        cp0 = pltpu.async_copy(rows_v, out_hbm.at[i0_v], sem)
        cp1 = pltpu.async_copy(rows_v, out_hbm.at[i1_v], sem)
        pltpu.sync_copy(i0_v, pos_hbm.at[pl.ds(base, ch)])
        pltpu.sync_copy(i1_v, pos_hbm.at[pl.ds(t + base, ch)])
        cp0.wait()
        cp1.wait()

    return k(flat, mrg, counts)


def _gather_rows(ys, pcat):
    n = pcat.shape[0]
    d = ys.shape[1]
    per_w = n // _NW
    ch = min(per_w, 64)
    nch = per_w // ch
    mesh = plsc.VectorSubcoreMesh(core_axis_name="c", subcore_axis_name="s")

    @functools.partial(
        pl.kernel,
        out_type=jax.ShapeDtypeStruct((n, d), ys.dtype),
        mesh=mesh,
        scratch_types=[
            pltpu.VMEM((ch,), I32),
            pltpu.VMEM((ch, d), ys.dtype),
            pltpu.SemaphoreType.DMA,
        ],
    )
    def k(ys_hbm, idx_hbm, out_hbm, idx_v, rows_v, sem):
        wid = lax.axis_index("c") * 16 + lax.axis_index("s")
        for c in range(nch):
            base = wid * per_w + c * ch
            pltpu.sync_copy(idx_hbm.at[pl.ds(base, ch)], idx_v)
            pltpu.async_copy(ys_hbm.at[idx_v], rows_v, sem).wait()
            pltpu.sync_copy(rows_v, out_hbm.at[pl.ds(base, ch)])

    return k(ys, pcat)


def kernel(x, Wr, Wg, Wu, Wd, Sg, Su, Sd):
    orig = x.shape
    d = orig[-1]
    flat = x.reshape(-1, d)
    t = flat.shape[0]
    e, m, _ = Wg.shape
    ns, ms, _ = Sg.shape
    ntb = t // _TB
    nt = (t * 2) // _TILE + e
    npad = nt * _TILE

    w0b, w1b, mrg, teb, countsb = pl.pallas_call(
        _router_body,
        grid=(ntb,),
        in_specs=[
            pl.BlockSpec((_TB, d), lambda i: (i, 0)),
            pl.BlockSpec((e, d), lambda i: (0, 0)),
        ],
        out_specs=[
            pl.BlockSpec((_TB, 8), lambda i: (i, 0)),
            pl.BlockSpec((_TB, 8), lambda i: (i, 0)),
            pl.BlockSpec((_TB, 128), lambda i: (i, 0)),
            pl.BlockSpec((nt, 8), lambda i: (0, 0)),
            pl.BlockSpec((1, e), lambda i: (0, 0)),
        ],
        out_shape=[
            jax.ShapeDtypeStruct((t, 8), F32),
            jax.ShapeDtypeStruct((t, 8), F32),
            jax.ShapeDtypeStruct((t, 128), I32),
            jax.ShapeDtypeStruct((nt, 8), I32),
            jax.ShapeDtypeStruct((1, e), I32),
        ],
        scratch_shapes=[pltpu.VMEM((1, e), I32)],
        compiler_params=pltpu.CompilerParams(
            dimension_semantics=("arbitrary",)),
    )(flat, Wr)

    sgc = Sg.reshape(ns * ms, d)
    suc = Su.reshape(ns * ms, d)
    shared = pl.pallas_call(
        _shared_body,
        grid=(2,),
        in_specs=[
            pl.BlockSpec((t // 2, d), lambda i: (i, 0)),
            pl.BlockSpec((ns * ms, d), lambda i: (0, 0)),
            pl.BlockSpec((ns * ms, d), lambda i: (0, 0)),
            pl.BlockSpec((ns, d, ms), lambda i: (0, 0, 0)),
        ],
        out_specs=pl.BlockSpec((t // 2, d), lambda i: (i, 0)),
        out_shape=jax.ShapeDtypeStruct((t, d), F32),
        compiler_params=pltpu.CompilerParams(
            dimension_semantics=("arbitrary",)),
    )(flat, sgc, suc, Sd)

    wgb, wub, wdb = pl.pallas_call(
        _cast_body,
        grid=(e,),
        in_specs=[
            pl.BlockSpec((1, m, d), lambda i: (i, 0, 0)),
            pl.BlockSpec((1, m, d), lambda i: (i, 0, 0)),
            pl.BlockSpec((1, d, m), lambda i: (i, 0, 0)),
        ],
        out_specs=[
            pl.BlockSpec((1, m, d), lambda i: (i, 0, 0)),
            pl.BlockSpec((1, m, d), lambda i: (i, 0, 0)),
            pl.BlockSpec((1, d, m), lambda i: (i, 0, 0)),
        ],
        out_shape=[
            jax.ShapeDtypeStruct((e, m, d), BF16),
            jax.ShapeDtypeStruct((e, m, d), BF16),
            jax.ShapeDtypeStruct((e, d, m), BF16),
        ],
        compiler_params=pltpu.CompilerParams(
            dimension_semantics=("arbitrary",)),
    )(Wg, Wu, Wd)

    xs, pos = _dispatch_tokens(flat, mrg, countsb, npad)

    ys = pl.pallas_call(
        _expert_body,
        grid_spec=pltpu.PrefetchScalarGridSpec(
            num_scalar_prefetch=1,
            grid=(nt,),
            in_specs=[
                pl.BlockSpec((_TILE, d), lambda j, eid: (j, 0)),
                pl.BlockSpec((e, m, d), lambda j, eid: (0, 0, 0)),
                pl.BlockSpec((e, m, d), lambda j, eid: (0, 0, 0)),
                pl.BlockSpec((e, d, m), lambda j, eid: (0, 0, 0)),
            ],
            out_specs=pl.BlockSpec((_TILE, d), lambda j, eid: (j, 0)),
        ),
        out_shape=jax.ShapeDtypeStruct((npad, d), F32),
        compiler_params=pltpu.CompilerParams(
            dimension_semantics=("arbitrary",)),
    )(teb, xs, wgb, wub, wdb)

    yg = _gather_rows(ys, pos)

    out = pl.pallas_call(
        _combine_body,
        grid=(ntb,),
        in_specs=[
            pl.BlockSpec((_TB, d), lambda i: (i, 0)),
            pl.BlockSpec((_TB, d), lambda i: (i, 0)),
            pl.BlockSpec((_TB, d), lambda i: (i + t // _TB, 0)),
            pl.BlockSpec((_TB, 8), lambda i: (i, 0)),
            pl.BlockSpec((_TB, 8), lambda i: (i, 0)),
        ],
        out_specs=pl.BlockSpec((_TB, d), lambda i: (i, 0)),
        out_shape=jax.ShapeDtypeStruct((t, d), F32),
        compiler_params=pltpu.CompilerParams(
            dimension_semantics=("arbitrary",)),
    )(shared, yg, yg, w0b, w1b)

    return out.reshape(orig)


# speedup vs baseline: 1.0452x; 1.0452x over previous
import dataclasses
import functools

import jax
import jax.numpy as jnp
from jax import lax
from jax.experimental import pallas as pl
from jax.experimental.pallas import tpu as pltpu
from jax.experimental.pallas import tpu_sc as plsc

F32 = jnp.float32
BF16 = jnp.bfloat16
I32 = jnp.int32

_TILE = 256
_TB = 256
_NW = 32
_LG2TILE = 8


def _sc_compiler_params():
    cp = pltpu.CompilerParams()
    if "needs_layout_passes" in pltpu.CompilerParams.__dataclass_fields__:
        cp = dataclasses.replace(cp, needs_layout_passes=False)
    return cp


def _router_body(x_ref, wr_ref, wg_ref, wu_ref, wd_ref,
                 w0_ref, w1_ref, mrg_ref, te_ref, counts_ref,
                 og_ref, ou_ref, od_ref, carry_ref):
    step = pl.program_id(0)

    @pl.when(step == 0)
    def _():
        carry_ref[...] = jnp.zeros_like(carry_ref)

    og_ref[...] = wg_ref[...].astype(BF16)
    ou_ref[...] = wu_ref[...].astype(BF16)
    od_ref[...] = wd_ref[...].astype(BF16)

    x = x_ref[...]
    logits = lax.dot_general(x, wr_ref[...], (((1,), (1,)), ((), ())),
                             preferred_element_type=F32)
    m = jnp.max(logits, axis=1, keepdims=True)
    p = jnp.exp(logits - m)
    probs = p / jnp.sum(p, axis=1, keepdims=True)
    ne = probs.shape[1]
    cols = lax.broadcasted_iota(I32, probs.shape, 1)
    w0 = jnp.max(probs, axis=1, keepdims=True)
    i0 = jnp.min(jnp.where(probs == w0, cols, ne), axis=1, keepdims=True)
    probs2 = jnp.where(cols == i0, jnp.float32(-1.0), probs)
    w1 = jnp.max(probs2, axis=1, keepdims=True)
    i1 = jnp.min(jnp.where(probs2 == w1, cols, ne), axis=1, keepdims=True)
    s = w0 + w1 + 1e-9
    w0_ref[...] = jnp.broadcast_to(w0 / s, w0_ref.shape)
    w1_ref[...] = jnp.broadcast_to(w1 / s, w1_ref.shape)

    tb = x.shape[0]
    oh0 = (lax.broadcasted_iota(I32, (tb, ne), 1) == i0).astype(F32)
    oh1 = (lax.broadcasted_iota(I32, (tb, ne), 1) == i1).astype(F32)
    ohs = jnp.concatenate([oh0, oh1], axis=0)
    ohf = ohs.astype(BF16)
    rows = lax.broadcasted_iota(I32, (2 * tb, 2 * tb), 0)
    colsq = lax.broadcasted_iota(I32, (2 * tb, 2 * tb), 1)
    tri = (rows >= colsq).astype(BF16)
    prefix = lax.dot_general(tri, ohf, (((1,), (0,)), ((), ())),
                             preferred_element_type=F32)
    carry = carry_ref[...].astype(F32)
    rank = jnp.sum((prefix + carry) * ohs, axis=1, keepdims=True) - 1.0
    rank = rank.astype(I32)
    r0 = rank[:tb]
    r1 = rank[tb:]

    lanes = lax.broadcasted_iota(I32, (tb, 128), 1)
    mrg = jnp.where(
        lanes < 8, jnp.broadcast_to(i0, (tb, 128)),
        jnp.where(lanes < 16, jnp.broadcast_to(i1, (tb, 128)),
                  jnp.where(lanes < 24, jnp.broadcast_to(r0, (tb, 128)),
                            jnp.broadcast_to(r1, (tb, 128)))))
    mrg_ref[...] = mrg

    new_counts = carry_ref[...] + jnp.sum(ohs, axis=0,
                                          keepdims=True).astype(I32)
    carry_ref[...] = new_counts
    counts_ref[...] = new_counts

    sizes = lax.shift_left(
        lax.shift_right_logical(new_counts + (_TILE - 1), _LG2TILE),
        _LG2TILE).astype(F32)
    tri16 = (lax.broadcasted_iota(I32, (ne, ne), 0)
             <= lax.broadcasted_iota(I32, (ne, ne), 1)).astype(F32)
    tcum = lax.dot_general(sizes, tri16, (((1,), (0,)), ((), ())),
                           preferred_element_type=F32)
    nt = te_ref.shape[0]
    tile_starts = lax.broadcasted_iota(I32, (nt, ne), 0) * _TILE
    te = jnp.sum((tile_starts >= jnp.broadcast_to(tcum.astype(I32),
                                                  (nt, ne))).astype(I32),
                 axis=1, keepdims=True)
    te = jnp.minimum(te, ne - 1)
    te_ref[...] = jnp.broadcast_to(te, te_ref.shape)


def _shared_body(x_ref, sg_ref, su_ref, sd_ref, o_ref):
    xb = x_ref[...].astype(BF16)
    g = lax.dot_general(xb, sg_ref[...].astype(BF16),
                        (((1,), (1,)), ((), ())), preferred_element_type=F32)
    u = lax.dot_general(xb, su_ref[...].astype(BF16),
                        (((1,), (1,)), ((), ())), preferred_element_type=F32)
    h = (g * jax.nn.sigmoid(g) * u).astype(BF16)
    ms = sd_ref.shape[2]
    y0 = lax.dot_general(h[:, :ms], sd_ref[0].astype(BF16),
                         (((1,), (1,)), ((), ())), preferred_element_type=F32)
    y1 = lax.dot_general(h[:, ms:], sd_ref[1].astype(BF16),
                         (((1,), (1,)), ((), ())), preferred_element_type=F32)
    o_ref[...] = y0 + y1


def _expert_body(eid_ref, xs_ref, wg_ref, wu_ref, wd_ref, o_ref):
    del eid_ref
    xb = xs_ref[...].astype(BF16)
    g = lax.dot_general(xb, wg_ref[0],
                        (((1,), (1,)), ((), ())), preferred_element_type=F32)
    u = lax.dot_general(xb, wu_ref[0],
                        (((1,), (1,)), ((), ())), preferred_element_type=F32)
    h = (g * jax.nn.sigmoid(g) * u).astype(BF16)
    o_ref[...] = lax.dot_general(h, wd_ref[0],
                                 (((1,), (1,)), ((), ())),
                                 preferred_element_type=F32)


def _combine_body(sh_ref, y0_ref, y1_ref, w0_ref, w1_ref, o_ref):
    o_ref[...] = (sh_ref[...]
                  + w0_ref[:, 0:1] * y0_ref[...]
                  + w1_ref[:, 0:1] * y1_ref[...])


def _dispatch_tokens(flat, mrg, counts, npad):
    t, d = flat.shape
    ne = counts.shape[1]
    ch = t // _NW
    mesh = plsc.VectorSubcoreMesh(core_axis_name="c", subcore_axis_name="s")

    @functools.partial(
        pl.kernel,
        out_type=[
            jax.ShapeDtypeStruct((npad, d), flat.dtype),
            jax.ShapeDtypeStruct((2 * t,), I32),
        ],
        mesh=mesh,
        scratch_types=[
            pltpu.VMEM((ne,), I32),
            pltpu.VMEM((ne,), I32),
            pltpu.VMEM((ch, 128), I32),
            pltpu.VMEM((ch,), I32),
            pltpu.VMEM((ch,), I32),
            pltpu.VMEM((ch, d), flat.dtype),
            pltpu.SemaphoreType.DMA,
            pltpu.SemaphoreType.DMA,
        ],
        compiler_params=_sc_compiler_params(),
    )
    def k(x_hbm, mrg_hbm, cnt_hbm, out_hbm, pos_hbm,
          cnt_v, gb_v, mrg_v, i0_v, i1_v, rows_v, sem, xsem):
        wid = lax.axis_index("c") * 16 + lax.axis_index("s")
        base = wid * ch
        xcp = pltpu.async_copy(x_hbm.at[pl.ds(base, ch)], rows_v, xsem)
        pltpu.sync_copy(cnt_hbm.at[0], cnt_v)
        pltpu.sync_copy(mrg_hbm.at[pl.ds(base, ch)], mrg_v)
        c = cnt_v[...]
        sizes = lax.shift_left(
            lax.shift_right_logical(c + (_TILE - 1), _LG2TILE), _LG2TILE)
        pref = plsc.cumsum(sizes)
        gb_v[...] = pref - sizes
        zero16 = lax.broadcasted_iota(I32, (16,), 0) * 0
        for j in range(ch // 16):
            rowsel = lax.broadcasted_iota(I32, (16,), 0) + (j * 16)
            e0v = plsc.load_gather(mrg_v, [rowsel, zero16])
            e1v = plsc.load_gather(mrg_v, [rowsel, zero16 + 8])
            r0v = plsc.load_gather(mrg_v, [rowsel, zero16 + 16])
            r1v = plsc.load_gather(mrg_v, [rowsel, zero16 + 24])
            i0_v[pl.ds(j * 16, 16)] = plsc.load_gather(gb_v, [e0v]) + r0v
            i1_v[pl.ds(j * 16, 16)] = plsc.load_gather(gb_v, [e1v]) + r1v
        xcp.wait()
        cp0 = pltpu.async_copy(rows_v, out_hbm.at[i0_v], sem)
        cp1 = pltpu.async_copy(rows_v, out_hbm.at[i1_v], sem)
        pltpu.sync_copy(i0_v, pos_hbm.at[pl.ds(base, ch)])
        pltpu.sync_copy(i1_v, pos_hbm.at[pl.ds(t + base, ch)])
        cp0.wait()
        cp1.wait()

    return k(flat, mrg, counts)


def _gather_rows(ys, pcat):
    n = pcat.shape[0]
    d = ys.shape[1]
    per_w = n // _NW
    ch = min(per_w, 64)
    nch = per_w // ch
    mesh = plsc.VectorSubcoreMesh(core_axis_name="c", subcore_axis_name="s")

    @functools.partial(
        pl.kernel,
        out_type=jax.ShapeDtypeStruct((n, d), ys.dtype),
        mesh=mesh,
        scratch_types=[
            pltpu.VMEM((ch,), I32),
            pltpu.VMEM((ch, d), ys.dtype),
            pltpu.SemaphoreType.DMA,
        ],
    )
    def k(ys_hbm, idx_hbm, out_hbm, idx_v, rows_v, sem):
        wid = lax.axis_index("c") * 16 + lax.axis_index("s")
        for c in range(nch):
            base = wid * per_w + c * ch
            pltpu.sync_copy(idx_hbm.at[pl.ds(base, ch)], idx_v)
            pltpu.async_copy(ys_hbm.at[idx_v], rows_v, sem).wait()
            pltpu.sync_copy(rows_v, out_hbm.at[pl.ds(base, ch)])

    return k(ys, pcat)


def kernel(x, Wr, Wg, Wu, Wd, Sg, Su, Sd):
    orig = x.shape
    d = orig[-1]
    flat = x.reshape(-1, d)
    t = flat.shape[0]
    e, m, _ = Wg.shape
    ns, ms, _ = Sg.shape
    ntb = t // _TB
    nt = (t * 2) // _TILE + e
    npad = nt * _TILE

    ept = e // ntb
    w0b, w1b, mrg, teb, countsb, wgb, wub, wdb = pl.pallas_call(
        _router_body,
        grid=(ntb,),
        in_specs=[
            pl.BlockSpec((_TB, d), lambda i: (i, 0)),
            pl.BlockSpec((e, d), lambda i: (0, 0)),
            pl.BlockSpec((ept, m, d), lambda i: (i, 0, 0)),
            pl.BlockSpec((ept, m, d), lambda i: (i, 0, 0)),
            pl.BlockSpec((ept, d, m), lambda i: (i, 0, 0)),
        ],
        out_specs=[
            pl.BlockSpec((_TB, 8), lambda i: (i, 0)),
            pl.BlockSpec((_TB, 8), lambda i: (i, 0)),
            pl.BlockSpec((_TB, 128), lambda i: (i, 0)),
            pl.BlockSpec((nt, 8), lambda i: (0, 0)),
            pl.BlockSpec((1, e), lambda i: (0, 0)),
            pl.BlockSpec((ept, m, d), lambda i: (i, 0, 0)),
            pl.BlockSpec((ept, m, d), lambda i: (i, 0, 0)),
            pl.BlockSpec((ept, d, m), lambda i: (i, 0, 0)),
        ],
        out_shape=[
            jax.ShapeDtypeStruct((t, 8), F32),
            jax.ShapeDtypeStruct((t, 8), F32),
            jax.ShapeDtypeStruct((t, 128), I32),
            jax.ShapeDtypeStruct((nt, 8), I32),
            jax.ShapeDtypeStruct((1, e), I32),
            jax.ShapeDtypeStruct((e, m, d), BF16),
            jax.ShapeDtypeStruct((e, m, d), BF16),
            jax.ShapeDtypeStruct((e, d, m), BF16),
        ],
        scratch_shapes=[pltpu.VMEM((1, e), I32)],
        compiler_params=pltpu.CompilerParams(
            dimension_semantics=("arbitrary",)),
    )(flat, Wr, Wg, Wu, Wd)

    sgc = Sg.reshape(ns * ms, d)
    suc = Su.reshape(ns * ms, d)
    shared = pl.pallas_call(
        _shared_body,
        grid=(2,),
        in_specs=[
            pl.BlockSpec((t // 2, d), lambda i: (i, 0)),
            pl.BlockSpec((ns * ms, d), lambda i: (0, 0)),
            pl.BlockSpec((ns * ms, d), lambda i: (0, 0)),
            pl.BlockSpec((ns, d, ms), lambda i: (0, 0, 0)),
        ],
        out_specs=pl.BlockSpec((t // 2, d), lambda i: (i, 0)),
        out_shape=jax.ShapeDtypeStruct((t, d), F32),
        compiler_params=pltpu.CompilerParams(
            dimension_semantics=("arbitrary",)),
    )(flat, sgc, suc, Sd)

    xs, pos = _dispatch_tokens(flat, mrg, countsb, npad)

    ys = pl.pallas_call(
        _expert_body,
        grid_spec=pltpu.PrefetchScalarGridSpec(
            num_scalar_prefetch=1,
            grid=(nt,),
            in_specs=[
                pl.BlockSpec((_TILE, d), lambda j, eid: (j, 0)),
                pl.BlockSpec((1, m, d), lambda j, eid: (eid[j, 0], 0, 0)),
                pl.BlockSpec((1, m, d), lambda j, eid: (eid[j, 0], 0, 0)),
                pl.BlockSpec((1, d, m), lambda j, eid: (eid[j, 0], 0, 0)),
            ],
            out_specs=pl.BlockSpec((_TILE, d), lambda j, eid: (j, 0)),
        ),
        out_shape=jax.ShapeDtypeStruct((npad, d), F32),
        compiler_params=pltpu.CompilerParams(
            dimension_semantics=("arbitrary",)),
    )(teb, xs, wgb, wub, wdb)

    yg = _gather_rows(ys, pos)

    out = pl.pallas_call(
        _combine_body,
        grid=(ntb,),
        in_specs=[
            pl.BlockSpec((_TB, d), lambda i: (i, 0)),
            pl.BlockSpec((_TB, d), lambda i: (i, 0)),
            pl.BlockSpec((_TB, d), lambda i: (i + t // _TB, 0)),
            pl.BlockSpec((_TB, 8), lambda i: (i, 0)),
            pl.BlockSpec((_TB, 8), lambda i: (i, 0)),
        ],
        out_specs=pl.BlockSpec((_TB, d), lambda i: (i, 0)),
        out_shape=jax.ShapeDtypeStruct((t, d), F32),
        compiler_params=pltpu.CompilerParams(
            dimension_semantics=("arbitrary",)),
    )(shared, yg, yg, w0b, w1b)

    return out.reshape(orig)


# speedup vs baseline: 1.1452x; 1.0957x over previous
import dataclasses
import functools

import jax
import jax.numpy as jnp
from jax import lax
from jax.experimental import pallas as pl
from jax.experimental.pallas import tpu as pltpu
from jax.experimental.pallas import tpu_sc as plsc

F32 = jnp.float32
BF16 = jnp.bfloat16
I32 = jnp.int32

_TILE = 256
_TB = 256
_NW = 32
_LG2TILE = 8


def _sc_compiler_params():
    cp = pltpu.CompilerParams()
    if "needs_layout_passes" in pltpu.CompilerParams.__dataclass_fields__:
        cp = dataclasses.replace(cp, needs_layout_passes=False)
    return cp


def _router_body(x_ref, wr_ref, w0_ref, w1_ref, mrg_ref, te_ref, counts_ref,
                 carry_ref):
    step = pl.program_id(0)

    @pl.when(step == 0)
    def _():
        carry_ref[...] = jnp.zeros_like(carry_ref)

    x = x_ref[...]
    logits = lax.dot_general(x, wr_ref[...], (((1,), (1,)), ((), ())),
                             preferred_element_type=F32)
    m = jnp.max(logits, axis=1, keepdims=True)
    p = jnp.exp(logits - m)
    probs = p / jnp.sum(p, axis=1, keepdims=True)
    ne = probs.shape[1]
    cols = lax.broadcasted_iota(I32, probs.shape, 1)
    w0 = jnp.max(probs, axis=1, keepdims=True)
    i0 = jnp.min(jnp.where(probs == w0, cols, ne), axis=1, keepdims=True)
    probs2 = jnp.where(cols == i0, jnp.float32(-1.0), probs)
    w1 = jnp.max(probs2, axis=1, keepdims=True)
    i1 = jnp.min(jnp.where(probs2 == w1, cols, ne), axis=1, keepdims=True)
    s = w0 + w1 + 1e-9
    w0_ref[...] = jnp.broadcast_to(w0 / s, w0_ref.shape)
    w1_ref[...] = jnp.broadcast_to(w1 / s, w1_ref.shape)

    tb = x.shape[0]
    oh0 = (lax.broadcasted_iota(I32, (tb, ne), 1) == i0).astype(F32)
    oh1 = (lax.broadcasted_iota(I32, (tb, ne), 1) == i1).astype(F32)
    ohs = jnp.concatenate([oh0, oh1], axis=0)
    ohf = ohs.astype(BF16)
    rows = lax.broadcasted_iota(I32, (2 * tb, 2 * tb), 0)
    colsq = lax.broadcasted_iota(I32, (2 * tb, 2 * tb), 1)
    tri = (rows >= colsq).astype(BF16)
    prefix = lax.dot_general(tri, ohf, (((1,), (0,)), ((), ())),
                             preferred_element_type=F32)
    carry = carry_ref[...].astype(F32)
    rank = jnp.sum((prefix + carry) * ohs, axis=1, keepdims=True) - 1.0
    rank = rank.astype(I32)
    r0 = rank[:tb]
    r1 = rank[tb:]

    lanes = lax.broadcasted_iota(I32, (tb, 128), 1)
    mrg = jnp.where(
        lanes < 8, jnp.broadcast_to(i0, (tb, 128)),
        jnp.where(lanes < 16, jnp.broadcast_to(i1, (tb, 128)),
                  jnp.where(lanes < 24, jnp.broadcast_to(r0, (tb, 128)),
                            jnp.broadcast_to(r1, (tb, 128)))))
    mrg_ref[...] = mrg

    new_counts = carry_ref[...] + jnp.sum(ohs, axis=0,
                                          keepdims=True).astype(I32)
    carry_ref[...] = new_counts
    counts_ref[...] = new_counts

    sizes = lax.shift_left(
        lax.shift_right_logical(new_counts + (_TILE - 1), _LG2TILE),
        _LG2TILE).astype(F32)
    tri16 = (lax.broadcasted_iota(I32, (ne, ne), 0)
             <= lax.broadcasted_iota(I32, (ne, ne), 1)).astype(F32)
    tcum = lax.dot_general(sizes, tri16, (((1,), (0,)), ((), ())),
                           preferred_element_type=F32)
    nt = te_ref.shape[0]
    tile_starts = lax.broadcasted_iota(I32, (nt, ne), 0) * _TILE
    te = jnp.sum((tile_starts >= jnp.broadcast_to(tcum.astype(I32),
                                                  (nt, ne))).astype(I32),
                 axis=1, keepdims=True)
    te = jnp.minimum(te, ne - 1)
    te_ref[...] = jnp.broadcast_to(te, te_ref.shape)


def _shared_body(x_ref, sg_ref, su_ref, sd_ref, o_ref):
    xb = x_ref[...].astype(BF16)
    g = lax.dot_general(xb, sg_ref[...].astype(BF16),
                        (((1,), (1,)), ((), ())), preferred_element_type=F32)
    u = lax.dot_general(xb, su_ref[...].astype(BF16),
                        (((1,), (1,)), ((), ())), preferred_element_type=F32)
    h = (g * jax.nn.sigmoid(g) * u).astype(BF16)
    ms = sd_ref.shape[2]
    y0 = lax.dot_general(h[:, :ms], sd_ref[0].astype(BF16),
                         (((1,), (1,)), ((), ())), preferred_element_type=F32)
    y1 = lax.dot_general(h[:, ms:], sd_ref[1].astype(BF16),
                         (((1,), (1,)), ((), ())), preferred_element_type=F32)
    o_ref[...] = y0 + y1


def _expert_body(eid_ref, xs_ref, wg_ref, wu_ref, wd_ref, o_ref):
    del eid_ref
    xb = xs_ref[...].astype(BF16)
    g = lax.dot_general(xb, wg_ref[0].astype(BF16),
                        (((1,), (1,)), ((), ())), preferred_element_type=F32)
    u = lax.dot_general(xb, wu_ref[0].astype(BF16),
                        (((1,), (1,)), ((), ())), preferred_element_type=F32)
    h = (g * jax.nn.sigmoid(g) * u).astype(BF16)
    o_ref[...] = lax.dot_general(h, wd_ref[0].astype(BF16),
                                 (((1,), (1,)), ((), ())),
                                 preferred_element_type=F32)


def _combine_body(sh_ref, y0_ref, y1_ref, w0_ref, w1_ref, o_ref):
    o_ref[...] = (sh_ref[...]
                  + w0_ref[:, 0:1] * y0_ref[...]
                  + w1_ref[:, 0:1] * y1_ref[...])


def _dispatch_tokens(flat, mrg, counts, npad):
    t, d = flat.shape
    ne = counts.shape[1]
    ch = t // _NW
    mesh = plsc.VectorSubcoreMesh(core_axis_name="c", subcore_axis_name="s")

    @functools.partial(
        pl.kernel,
        out_type=[
            jax.ShapeDtypeStruct((npad, d), flat.dtype),
            jax.ShapeDtypeStruct((2 * t,), I32),
        ],
        mesh=mesh,
        scratch_types=[
            pltpu.VMEM((ne,), I32),
            pltpu.VMEM((ne,), I32),
            pltpu.VMEM((ch, 128), I32),
            pltpu.VMEM((ch,), I32),
            pltpu.VMEM((ch,), I32),
            pltpu.VMEM((ch, d), flat.dtype),
            pltpu.SemaphoreType.DMA,
            pltpu.SemaphoreType.DMA,
        ],
        compiler_params=_sc_compiler_params(),
    )
    def k(x_hbm, mrg_hbm, cnt_hbm, out_hbm, pos_hbm,
          cnt_v, gb_v, mrg_v, i0_v, i1_v, rows_v, sem, xsem):
        wid = lax.axis_index("c") * 16 + lax.axis_index("s")
        base = wid * ch
        xcp = pltpu.async_copy(x_hbm.at[pl.ds(base, ch)], rows_v, xsem)
        pltpu.sync_copy(cnt_hbm.at[0], cnt_v)
        pltpu.sync_copy(mrg_hbm.at[pl.ds(base, ch)], mrg_v)
        c = cnt_v[...]
        sizes = lax.shift_left(
            lax.shift_right_logical(c + (_TILE - 1), _LG2TILE), _LG2TILE)
        pref = plsc.cumsum(sizes)
        gb_v[...] = pref - sizes
        zero16 = lax.broadcasted_iota(I32, (16,), 0) * 0
        for j in range(ch // 16):
            rowsel = lax.broadcasted_iota(I32, (16,), 0) + (j * 16)
            e0v = plsc.load_gather(mrg_v, [rowsel, zero16])
            e1v = plsc.load_gather(mrg_v, [rowsel, zero16 + 8])
            r0v = plsc.load_gather(mrg_v, [rowsel, zero16 + 16])
            r1v = plsc.load_gather(mrg_v, [rowsel, zero16 + 24])
            i0_v[pl.ds(j * 16, 16)] = plsc.load_gather(gb_v, [e0v]) + r0v
            i1_v[pl.ds(j * 16, 16)] = plsc.load_gather(gb_v, [e1v]) + r1v
        xcp.wait()
        cp0 = pltpu.async_copy(rows_v, out_hbm.at[i0_v], sem)
        cp1 = pltpu.async_copy(rows_v, out_hbm.at[i1_v], sem)
        pltpu.sync_copy(i0_v, pos_hbm.at[pl.ds(base, ch)])
        pltpu.sync_copy(i1_v, pos_hbm.at[pl.ds(t + base, ch)])
        cp0.wait()
        cp1.wait()

    return k(flat, mrg, counts)


def _gather_rows(ys, pcat):
    n = pcat.shape[0]
    d = ys.shape[1]
    per_w = n // _NW
    ch = min(per_w, 64)
    nch = per_w // ch
    mesh = plsc.VectorSubcoreMesh(core_axis_name="c", subcore_axis_name="s")

    @functools.partial(
        pl.kernel,
        out_type=jax.ShapeDtypeStruct((n, d), ys.dtype),
        mesh=mesh,
        scratch_types=[
            pltpu.VMEM((ch,), I32),
            pltpu.VMEM((ch, d), ys.dtype),
            pltpu.SemaphoreType.DMA,
        ],
    )
    def k(ys_hbm, idx_hbm, out_hbm, idx_v, rows_v, sem):
        wid = lax.axis_index("c") * 16 + lax.axis_index("s")
        for c in range(nch):
            base = wid * per_w + c * ch
            pltpu.sync_copy(idx_hbm.at[pl.ds(base, ch)], idx_v)
            pltpu.async_copy(ys_hbm.at[idx_v], rows_v, sem).wait()
            pltpu.sync_copy(rows_v, out_hbm.at[pl.ds(base, ch)])

    return k(ys, pcat)


def kernel(x, Wr, Wg, Wu, Wd, Sg, Su, Sd):
    orig = x.shape
    d = orig[-1]
    flat = x.reshape(-1, d)
    t = flat.shape[0]
    e, m, _ = Wg.shape
    ns, ms, _ = Sg.shape
    ntb = t // _TB
    nt = (t * 2) // _TILE + e
    npad = nt * _TILE

    w0b, w1b, mrg, teb, countsb = pl.pallas_call(
        _router_body,
        grid=(ntb,),
        in_specs=[
            pl.BlockSpec((_TB, d), lambda i: (i, 0)),
            pl.BlockSpec((e, d), lambda i: (0, 0)),
        ],
        out_specs=[
            pl.BlockSpec((_TB, 8), lambda i: (i, 0)),
            pl.BlockSpec((_TB, 8), lambda i: (i, 0)),
            pl.BlockSpec((_TB, 128), lambda i: (i, 0)),
            pl.BlockSpec((nt, 8), lambda i: (0, 0)),
            pl.BlockSpec((1, e), lambda i: (0, 0)),
        ],
        out_shape=[
            jax.ShapeDtypeStruct((t, 8), F32),
            jax.ShapeDtypeStruct((t, 8), F32),
            jax.ShapeDtypeStruct((t, 128), I32),
            jax.ShapeDtypeStruct((nt, 8), I32),
            jax.ShapeDtypeStruct((1, e), I32),
        ],
        scratch_shapes=[pltpu.VMEM((1, e), I32)],
        compiler_params=pltpu.CompilerParams(
            dimension_semantics=("arbitrary",)),
    )(flat, Wr)

    sgc = Sg.reshape(ns * ms, d)
    suc = Su.reshape(ns * ms, d)
    shared = pl.pallas_call(
        _shared_body,
        grid=(2,),
        in_specs=[
            pl.BlockSpec((t // 2, d), lambda i: (i, 0)),
            pl.BlockSpec((ns * ms, d), lambda i: (0, 0)),
            pl.BlockSpec((ns * ms, d), lambda i: (0, 0)),
            pl.BlockSpec((ns, d, ms), lambda i: (0, 0, 0)),
        ],
        out_specs=pl.BlockSpec((t // 2, d), lambda i: (i, 0)),
        out_shape=jax.ShapeDtypeStruct((t, d), F32),
        compiler_params=pltpu.CompilerParams(
            dimension_semantics=("arbitrary",)),
    )(flat, sgc, suc, Sd)

    xs, pos = _dispatch_tokens(flat, mrg, countsb, npad)

    ys = pl.pallas_call(
        _expert_body,
        grid_spec=pltpu.PrefetchScalarGridSpec(
            num_scalar_prefetch=1,
            grid=(nt,),
            in_specs=[
                pl.BlockSpec((_TILE, d), lambda j, eid: (j, 0)),
                pl.BlockSpec((1, m, d), lambda j, eid: (eid[j, 0], 0, 0)),
                pl.BlockSpec((1, m, d), lambda j, eid: (eid[j, 0], 0, 0)),
                pl.BlockSpec((1, d, m), lambda j, eid: (eid[j, 0], 0, 0)),
            ],
            out_specs=pl.BlockSpec((_TILE, d), lambda j, eid: (j, 0)),
        ),
        out_shape=jax.ShapeDtypeStruct((npad, d), F32),
        compiler_params=pltpu.CompilerParams(
            dimension_semantics=("arbitrary",)),
    )(teb, xs, Wg, Wu, Wd)

    yg = _gather_rows(ys, pos)

    out = pl.pallas_call(
        _combine_body,
        grid=(ntb,),
        in_specs=[
            pl.BlockSpec((_TB, d), lambda i: (i, 0)),
            pl.BlockSpec((_TB, d), lambda i: (i, 0)),
            pl.BlockSpec((_TB, d), lambda i: (i + t // _TB, 0)),
            pl.BlockSpec((_TB, 8), lambda i: (i, 0)),
            pl.BlockSpec((_TB, 8), lambda i: (i, 0)),
        ],
        out_specs=pl.BlockSpec((_TB, d), lambda i: (i, 0)),
        out_shape=jax.ShapeDtypeStruct((t, d), F32),
        compiler_params=pltpu.CompilerParams(
            dimension_semantics=("arbitrary",)),
    )(shared, yg, yg, w0b, w1b)

    return out.reshape(orig)


# speedup vs baseline: 1.1470x; 1.0016x over previous
import dataclasses
import functools

import jax
import jax.numpy as jnp
from jax import lax
from jax.experimental import pallas as pl
from jax.experimental.pallas import tpu as pltpu
from jax.experimental.pallas import tpu_sc as plsc

F32 = jnp.float32
BF16 = jnp.bfloat16
I32 = jnp.int32

_TILE = 256
_TB = 256
_NW = 32
_LG2TILE = 8


def _sc_compiler_params():
    cp = pltpu.CompilerParams()
    if "needs_layout_passes" in pltpu.CompilerParams.__dataclass_fields__:
        cp = dataclasses.replace(cp, needs_layout_passes=False)
    return cp


def _router_body(x_ref, wr_ref, w0_ref, w1_ref, mrg_ref, te_ref, counts_ref,
                 carry_ref, tri_ref):
    step = pl.program_id(0)

    @pl.when(step == 0)
    def _():
        carry_ref[...] = jnp.zeros_like(carry_ref)
        n2 = tri_ref.shape[0]
        rows = lax.broadcasted_iota(I32, (n2, n2), 0)
        colsq = lax.broadcasted_iota(I32, (n2, n2), 1)
        tri_ref[...] = (rows >= colsq).astype(BF16)

    x = x_ref[...]
    logits = lax.dot_general(x, wr_ref[...], (((1,), (1,)), ((), ())),
                             preferred_element_type=F32)
    m = jnp.max(logits, axis=1, keepdims=True)
    p = jnp.exp(logits - m)
    probs = p / jnp.sum(p, axis=1, keepdims=True)
    ne = probs.shape[1]
    cols = lax.broadcasted_iota(I32, probs.shape, 1)
    w0 = jnp.max(probs, axis=1, keepdims=True)
    i0 = jnp.min(jnp.where(probs == w0, cols, ne), axis=1, keepdims=True)
    probs2 = jnp.where(cols == i0, jnp.float32(-1.0), probs)
    w1 = jnp.max(probs2, axis=1, keepdims=True)
    i1 = jnp.min(jnp.where(probs2 == w1, cols, ne), axis=1, keepdims=True)
    s = w0 + w1 + 1e-9
    w0_ref[...] = jnp.broadcast_to(w0 / s, w0_ref.shape)
    w1_ref[...] = jnp.broadcast_to(w1 / s, w1_ref.shape)

    tb = x.shape[0]
    oh0 = (lax.broadcasted_iota(I32, (tb, ne), 1) == i0).astype(F32)
    oh1 = (lax.broadcasted_iota(I32, (tb, ne), 1) == i1).astype(F32)
    ohs = jnp.concatenate([oh0, oh1], axis=0)
    ohf = ohs.astype(BF16)
    prefix = lax.dot_general(tri_ref[...], ohf, (((1,), (0,)), ((), ())),
                             preferred_element_type=F32)
    carry = carry_ref[...].astype(F32)
    rank = jnp.sum((prefix + carry) * ohs, axis=1, keepdims=True) - 1.0
    rank = rank.astype(I32)
    r0 = rank[:tb]
    r1 = rank[tb:]

    lanes = lax.broadcasted_iota(I32, (tb, 128), 1)
    mrg = jnp.where(
        lanes < 8, jnp.broadcast_to(i0, (tb, 128)),
        jnp.where(lanes < 16, jnp.broadcast_to(i1, (tb, 128)),
                  jnp.where(lanes < 24, jnp.broadcast_to(r0, (tb, 128)),
                            jnp.broadcast_to(r1, (tb, 128)))))
    mrg_ref[...] = mrg

    new_counts = carry_ref[...] + jnp.sum(ohs, axis=0,
                                          keepdims=True).astype(I32)
    carry_ref[...] = new_counts
    counts_ref[...] = new_counts

    sizes = lax.shift_left(
        lax.shift_right_logical(new_counts + (_TILE - 1), _LG2TILE),
        _LG2TILE).astype(F32)
    tri16 = (lax.broadcasted_iota(I32, (ne, ne), 0)
             <= lax.broadcasted_iota(I32, (ne, ne), 1)).astype(F32)
    tcum = lax.dot_general(sizes, tri16, (((1,), (0,)), ((), ())),
                           preferred_element_type=F32)
    nt = te_ref.shape[0]
    tile_starts = lax.broadcasted_iota(I32, (nt, ne), 0) * _TILE
    te = jnp.sum((tile_starts >= jnp.broadcast_to(tcum.astype(I32),
                                                  (nt, ne))).astype(I32),
                 axis=1, keepdims=True)
    te = jnp.minimum(te, ne - 1)
    te_ref[...] = jnp.broadcast_to(te, te_ref.shape)


def _shared_body(x_ref, sg_ref, su_ref, sd_ref, o_ref):
    xb = x_ref[...].astype(BF16)
    g = lax.dot_general(xb, sg_ref[...].astype(BF16),
                        (((1,), (1,)), ((), ())), preferred_element_type=F32)
    u = lax.dot_general(xb, su_ref[...].astype(BF16),
                        (((1,), (1,)), ((), ())), preferred_element_type=F32)
    h = (g * jax.nn.sigmoid(g) * u).astype(BF16)
    ms = sd_ref.shape[2]
    y0 = lax.dot_general(h[:, :ms], sd_ref[0].astype(BF16),
                         (((1,), (1,)), ((), ())), preferred_element_type=F32)
    y1 = lax.dot_general(h[:, ms:], sd_ref[1].astype(BF16),
                         (((1,), (1,)), ((), ())), preferred_element_type=F32)
    o_ref[...] = y0 + y1


def _expert_body(eid_ref, xs_ref, wg_ref, wu_ref, wd_ref, o_ref):
    del eid_ref
    xb = xs_ref[...].astype(BF16)
    g = lax.dot_general(xb, wg_ref[0].astype(BF16),
                        (((1,), (1,)), ((), ())), preferred_element_type=F32)
    u = lax.dot_general(xb, wu_ref[0].astype(BF16),
                        (((1,), (1,)), ((), ())), preferred_element_type=F32)
    h = (g * jax.nn.sigmoid(g) * u).astype(BF16)
    o_ref[...] = lax.dot_general(h, wd_ref[0].astype(BF16),
                                 (((1,), (1,)), ((), ())),
                                 preferred_element_type=F32)


def _combine_body(sh_ref, y0_ref, y1_ref, w0_ref, w1_ref, o_ref):
    o_ref[...] = (sh_ref[...]
                  + w0_ref[:, 0:1] * y0_ref[...]
                  + w1_ref[:, 0:1] * y1_ref[...])


def _dispatch_tokens(flat, mrg, counts, npad):
    t, d = flat.shape
    ne = counts.shape[1]
    ch = t // _NW
    mesh = plsc.VectorSubcoreMesh(core_axis_name="c", subcore_axis_name="s")

    @functools.partial(
        pl.kernel,
        out_type=[
            jax.ShapeDtypeStruct((npad, d), flat.dtype),
            jax.ShapeDtypeStruct((2 * t,), I32),
        ],
        mesh=mesh,
        scratch_types=[
            pltpu.VMEM((ne,), I32),
            pltpu.VMEM((ne,), I32),
            pltpu.VMEM((ch, 128), I32),
            pltpu.VMEM((ch,), I32),
            pltpu.VMEM((ch,), I32),
            pltpu.VMEM((ch, d), flat.dtype),
            pltpu.SemaphoreType.DMA,
            pltpu.SemaphoreType.DMA,
        ],
        compiler_params=_sc_compiler_params(),
    )
    def k(x_hbm, mrg_hbm, cnt_hbm, out_hbm, pos_hbm,
          cnt_v, gb_v, mrg_v, i0_v, i1_v, rows_v, sem, xsem):
        wid = lax.axis_index("c") * 16 + lax.axis_index("s")
        base = wid * ch
        xcp = pltpu.async_copy(x_hbm.at[pl.ds(base, ch)], rows_v, xsem)
        pltpu.sync_copy(cnt_hbm.at[0], cnt_v)
        pltpu.sync_copy(mrg_hbm.at[pl.ds(base, ch)], mrg_v)
        c = cnt_v[...]
        sizes = lax.shift_left(
            lax.shift_right_logical(c + (_TILE - 1), _LG2TILE), _LG2TILE)
        pref = plsc.cumsum(sizes)
        gb_v[...] = pref - sizes
        zero16 = lax.broadcasted_iota(I32, (16,), 0) * 0
        for j in range(ch // 16):
            rowsel = lax.broadcasted_iota(I32, (16,), 0) + (j * 16)
            e0v = plsc.load_gather(mrg_v, [rowsel, zero16])
            e1v = plsc.load_gather(mrg_v, [rowsel, zero16 + 8])
            r0v = plsc.load_gather(mrg_v, [rowsel, zero16 + 16])
            r1v = plsc.load_gather(mrg_v, [rowsel, zero16 + 24])
            i0_v[pl.ds(j * 16, 16)] = plsc.load_gather(gb_v, [e0v]) + r0v
            i1_v[pl.ds(j * 16, 16)] = plsc.load_gather(gb_v, [e1v]) + r1v
        xcp.wait()
        cp0 = pltpu.async_copy(rows_v, out_hbm.at[i0_v], sem)
        cp1 = pltpu.async_copy(rows_v, out_hbm.at[i1_v], sem)
        pltpu.sync_copy(i0_v, pos_hbm.at[pl.ds(base, ch)])
        pltpu.sync_copy(i1_v, pos_hbm.at[pl.ds(t + base, ch)])
        cp0.wait()
        cp1.wait()

    return k(flat, mrg, counts)


def _gather_rows(ys, pcat):
    n = pcat.shape[0]
    d = ys.shape[1]
    per_w = n // _NW
    ch = min(per_w, 64)
    nch = per_w // ch
    mesh = plsc.VectorSubcoreMesh(core_axis_name="c", subcore_axis_name="s")

    @functools.partial(
        pl.kernel,
        out_type=jax.ShapeDtypeStruct((n, d), ys.dtype),
        mesh=mesh,
        scratch_types=[
            pltpu.VMEM((ch,), I32),
            pltpu.VMEM((ch, d), ys.dtype),
            pltpu.SemaphoreType.DMA,
        ],
    )
    def k(ys_hbm, idx_hbm, out_hbm, idx_v, rows_v, sem):
        wid = lax.axis_index("c") * 16 + lax.axis_index("s")
        for c in range(nch):
            base = wid * per_w + c * ch
            pltpu.sync_copy(idx_hbm.at[pl.ds(base, ch)], idx_v)
            pltpu.async_copy(ys_hbm.at[idx_v], rows_v, sem).wait()
            pltpu.sync_copy(rows_v, out_hbm.at[pl.ds(base, ch)])

    return k(ys, pcat)


def kernel(x, Wr, Wg, Wu, Wd, Sg, Su, Sd):
    orig = x.shape
    d = orig[-1]
    flat = x.reshape(-1, d)
    t = flat.shape[0]
    e, m, _ = Wg.shape
    ns, ms, _ = Sg.shape
    ntb = t // _TB
    nt = (t * 2) // _TILE + e
    npad = nt * _TILE

    w0b, w1b, mrg, teb, countsb = pl.pallas_call(
        _router_body,
        grid=(ntb,),
        in_specs=[
            pl.BlockSpec((_TB, d), lambda i: (i, 0)),
            pl.BlockSpec((e, d), lambda i: (0, 0)),
        ],
        out_specs=[
            pl.BlockSpec((_TB, 8), lambda i: (i, 0)),
            pl.BlockSpec((_TB, 8), lambda i: (i, 0)),
            pl.BlockSpec((_TB, 128), lambda i: (i, 0)),
            pl.BlockSpec((nt, 8), lambda i: (0, 0)),
            pl.BlockSpec((1, e), lambda i: (0, 0)),
        ],
        out_shape=[
            jax.ShapeDtypeStruct((t, 8), F32),
            jax.ShapeDtypeStruct((t, 8), F32),
            jax.ShapeDtypeStruct((t, 128), I32),
            jax.ShapeDtypeStruct((nt, 8), I32),
            jax.ShapeDtypeStruct((1, e), I32),
        ],
        scratch_shapes=[pltpu.VMEM((1, e), I32),
                        pltpu.VMEM((2 * _TB, 2 * _TB), BF16)],
        compiler_params=pltpu.CompilerParams(
            dimension_semantics=("arbitrary",)),
    )(flat, Wr)

    sgc = Sg.reshape(ns * ms, d)
    suc = Su.reshape(ns * ms, d)
    shared = pl.pallas_call(
        _shared_body,
        grid=(2,),
        in_specs=[
            pl.BlockSpec((t // 2, d), lambda i: (i, 0)),
            pl.BlockSpec((ns * ms, d), lambda i: (0, 0)),
            pl.BlockSpec((ns * ms, d), lambda i: (0, 0)),
            pl.BlockSpec((ns, d, ms), lambda i: (0, 0, 0)),
        ],
        out_specs=pl.BlockSpec((t // 2, d), lambda i: (i, 0)),
        out_shape=jax.ShapeDtypeStruct((t, d), F32),
        compiler_params=pltpu.CompilerParams(
            dimension_semantics=("arbitrary",)),
    )(flat, sgc, suc, Sd)

    xs, pos = _dispatch_tokens(flat, mrg, countsb, npad)

    ys = pl.pallas_call(
        _expert_body,
        grid_spec=pltpu.PrefetchScalarGridSpec(
            num_scalar_prefetch=1,
            grid=(nt,),
            in_specs=[
                pl.BlockSpec((_TILE, d), lambda j, eid: (j, 0)),
                pl.BlockSpec((1, m, d), lambda j, eid: (eid[j, 0], 0, 0)),
                pl.BlockSpec((1, m, d), lambda j, eid: (eid[j, 0], 0, 0)),
                pl.BlockSpec((1, d, m), lambda j, eid: (eid[j, 0], 0, 0)),
            ],
            out_specs=pl.BlockSpec((_TILE, d), lambda j, eid: (j, 0)),
        ),
        out_shape=jax.ShapeDtypeStruct((npad, d), F32),
        compiler_params=pltpu.CompilerParams(
            dimension_semantics=("arbitrary",)),
    )(teb, xs, Wg, Wu, Wd)

    yg = _gather_rows(ys, pos)

    out = pl.pallas_call(
        _combine_body,
        grid=(ntb,),
        in_specs=[
            pl.BlockSpec((_TB, d), lambda i: (i, 0)),
            pl.BlockSpec((_TB, d), lambda i: (i, 0)),
            pl.BlockSpec((_TB, d), lambda i: (i + t // _TB, 0)),
            pl.BlockSpec((_TB, 8), lambda i: (i, 0)),
            pl.BlockSpec((_TB, 8), lambda i: (i, 0)),
        ],
        out_specs=pl.BlockSpec((_TB, d), lambda i: (i, 0)),
        out_shape=jax.ShapeDtypeStruct((t, d), F32),
        compiler_params=pltpu.CompilerParams(
            dimension_semantics=("arbitrary",)),
    )(shared, yg, yg, w0b, w1b)

    return out.reshape(orig)
